# baseline (device time: 39713 ns/iter reference)
import jax
import jax.numpy as jnp
from jax import lax
from jax.experimental import pallas as pl
from jax.experimental.pallas import tpu as pltpu

M = 1024
N = 1024
SIZES = (256, 128, 64)


def kernel(A, B):
    def body(a_ref, b_ref, out_ref, acc_ref, sbuf_ref, rbuf_ref,
             send_sems, recv_sems):
        p = lax.axis_index("i")
        bit0 = p % 2
        bit1 = (p // 2) % 2
        bit2 = (p // 4) % 2

        barrier = pltpu.get_barrier_semaphore()
        for mask in (3, 1, 4):
            pl.semaphore_signal(
                barrier,
                inc=1,
                device_id=(lax.bitwise_xor(p, mask),),
                device_id_type=pl.DeviceIdType.MESH,
            )
        pl.semaphore_wait(barrier, 3)

        parts = []
        for base, masks, bits, soff, sem0 in (
            (0, (3, 1, 4), (bit1, bit0, bit2), (0, 256, 384), 0),
            (512, (4, 3, 1), (bit2, bit1, bit0), (448, 704, 832), 6),
        ):
            keep, send = [], []
            cur = base
            for t in range(3):
                sz, b = SIZES[t], bits[t]
                keep.append(cur + b * sz)
                send.append(cur + (1 - b) * sz)
                cur = keep[t]
            parts.append(dict(
                masks=masks, keep=keep, send=send, soff=soff, sem0=sem0,
            ))

        inflight = {}

        def rs_start(pi, t):
            pt = parts[pi]
            sz, off = SIZES[t], pt["soff"][t]
            sbuf_ref[pl.ds(off, sz), :] = acc_ref[
                pl.ds(pt["send"][t], sz), :
            ].astype(jnp.bfloat16)
            rdma = pltpu.make_async_remote_copy(
                src_ref=sbuf_ref.at[pl.ds(off, sz), :],
                dst_ref=rbuf_ref.at[pl.ds(off, sz), :],
                send_sem=send_sems.at[pt["sem0"] + t],
                recv_sem=recv_sems.at[pt["sem0"] + t],
                device_id=(lax.bitwise_xor(p, pt["masks"][t]),),
                device_id_type=pl.DeviceIdType.MESH,
            )
            rdma.start()
            inflight[("rs", pi, t)] = rdma

        def rs_finish(pi, t):
            pt = parts[pi]
            sz, off = SIZES[t], pt["soff"][t]
            inflight.pop(("rs", pi, t)).wait()
            acc_ref[pl.ds(pt["keep"][t], sz), :] = (
                acc_ref[pl.ds(pt["keep"][t], sz), :]
                + rbuf_ref[pl.ds(off, sz), :].astype(jnp.float32)
            )

        def ag_start(pi, t):
            pt = parts[pi]
            sz, sbase = SIZES[2 - t], pt["keep"][2 - t]
            rdma = pltpu.make_async_remote_copy(
                src_ref=out_ref.at[pl.ds(sbase, sz), :],
                dst_ref=out_ref.at[pl.ds(sbase, sz), :],
                send_sem=send_sems.at[pt["sem0"] + 3 + t],
                recv_sem=recv_sems.at[pt["sem0"] + 3 + t],
                device_id=(lax.bitwise_xor(p, pt["masks"][2 - t]),),
                device_id_type=pl.DeviceIdType.MESH,
            )
            rdma.start()
            inflight[("ag", pi, t)] = rdma

        acc_ref[0:512, :] = jnp.dot(
            a_ref[0:512, :].astype(jnp.bfloat16),
            b_ref[:, :].astype(jnp.bfloat16),
            preferred_element_type=jnp.float32,
        )
        rs_start(0, 0)
        acc_ref[512:M, :] = jnp.dot(
            a_ref[512:M, :].astype(jnp.bfloat16),
            b_ref[:, :].astype(jnp.bfloat16),
            preferred_element_type=jnp.float32,
        )
        rs_start(1, 0)

        for t in range(3):
            for pi in (0, 1):
                rs_finish(pi, t)
                if t < 2:
                    rs_start(pi, t + 1)
                else:
                    k3 = parts[pi]["keep"][2]
                    out_ref[pl.ds(k3, 64), :] = jnp.maximum(
                        acc_ref[pl.ds(k3, 64), :], 0.0
                    ).astype(jnp.bfloat16)
                    ag_start(pi, 0)

        for t in range(3):
            for pi in (0, 1):
                inflight.pop(("ag", pi, t)).wait()
                if t < 2:
                    ag_start(pi, t + 1)

    return pl.pallas_call(
        body,
        out_shape=jax.ShapeDtypeStruct((M, N), jnp.bfloat16),
        in_specs=[
            pl.BlockSpec(memory_space=pltpu.VMEM),
            pl.BlockSpec(memory_space=pltpu.VMEM),
        ],
        out_specs=pl.BlockSpec(memory_space=pltpu.VMEM),
        scratch_shapes=[
            pltpu.VMEM((M, N), jnp.float32),
            pltpu.VMEM((896, N), jnp.bfloat16),
            pltpu.VMEM((896, N), jnp.bfloat16),
            pltpu.SemaphoreType.DMA((12,)),
            pltpu.SemaphoreType.DMA((12,)),
        ],
        compiler_params=pltpu.CompilerParams(collective_id=0),
    )(A, B)


# device time: 34601 ns/iter; 1.1477x vs baseline; 1.1477x over previous
import jax
import jax.numpy as jnp
from jax import lax
from jax.experimental import pallas as pl
from jax.experimental.pallas import tpu as pltpu

M = 1024
N = 1024


def kernel(A, B):
    def body(a_ref, b_ref, out_ref, acc_ref, sbuf_ref, rbuf_ref,
             send_sems, recv_sems):
        p = lax.axis_index("i")
        bit0 = p % 2
        bit1 = (p // 2) % 2
        bit2 = (p // 4) % 2

        barrier = pltpu.get_barrier_semaphore()
        for mask in (3, 1, 4):
            pl.semaphore_signal(
                barrier,
                inc=1,
                device_id=(lax.bitwise_xor(p, mask),),
                device_id_type=pl.DeviceIdType.MESH,
            )
        pl.semaphore_wait(barrier, 3)

        parts = []
        for base, rows, masks, bits, soff, sem0 in (
            (0, 384, (3, 1, 4), (bit1, bit0, bit2), (0, 192, 288), 0),
            (384, 384, (1, 4, 3), (bit0 ^ bit1, bit2, bit0), (336, 528, 624), 6),
            (768, 256, (4, 3, 1), (bit2, bit1, bit0), (672, 800, 864), 12),
        ):
            sizes = (rows // 2, rows // 4, rows // 8)
            keep, send = [], []
            cur = base
            for t in range(3):
                sz, b = sizes[t], bits[t]
                keep.append(cur + b * sz)
                send.append(cur + (1 - b) * sz)
                cur = keep[t]
            parts.append(dict(
                rows=rows, sizes=sizes, masks=masks, keep=keep,
                send=send, soff=soff, sem0=sem0,
            ))

        inflight = {}

        def rs_start(pi, t):
            pt = parts[pi]
            sz, off = pt["sizes"][t], pt["soff"][t]
            sbuf_ref[pl.ds(off, sz), :] = acc_ref[
                pl.ds(pt["send"][t], sz), :
            ].astype(jnp.bfloat16)
            rdma = pltpu.make_async_remote_copy(
                src_ref=sbuf_ref.at[pl.ds(off, sz), :],
                dst_ref=rbuf_ref.at[pl.ds(off, sz), :],
                send_sem=send_sems.at[pt["sem0"] + t],
                recv_sem=recv_sems.at[pt["sem0"] + t],
                device_id=(lax.bitwise_xor(p, pt["masks"][t]),),
                device_id_type=pl.DeviceIdType.MESH,
            )
            rdma.start()
            inflight[("rs", pi, t)] = rdma

        def rs_finish(pi, t):
            pt = parts[pi]
            sz, off = pt["sizes"][t], pt["soff"][t]
            inflight.pop(("rs", pi, t)).wait()
            summed = (
                acc_ref[pl.ds(pt["keep"][t], sz), :]
                + rbuf_ref[pl.ds(off, sz), :].astype(jnp.float32)
            )
            if t < 2:
                acc_ref[pl.ds(pt["keep"][t], sz), :] = summed
            else:
                out_ref[pl.ds(pt["keep"][t], sz), :] = jnp.maximum(
                    summed, 0.0
                ).astype(jnp.bfloat16)

        def ag_start(pi, t):
            pt = parts[pi]
            sz, sbase = pt["sizes"][2 - t], pt["keep"][2 - t]
            rdma = pltpu.make_async_remote_copy(
                src_ref=out_ref.at[pl.ds(sbase, sz), :],
                dst_ref=out_ref.at[pl.ds(sbase, sz), :],
                send_sem=send_sems.at[pt["sem0"] + 3 + t],
                recv_sem=recv_sems.at[pt["sem0"] + 3 + t],
                device_id=(lax.bitwise_xor(p, pt["masks"][2 - t]),),
                device_id_type=pl.DeviceIdType.MESH,
            )
            rdma.start()
            inflight[("ag", pi, t)] = rdma

        for pi, (lo, hi) in enumerate(((0, 384), (384, 768), (768, M))):
            acc_ref[lo:hi, :] = jnp.dot(
                a_ref[lo:hi, :].astype(jnp.bfloat16),
                b_ref[:, :].astype(jnp.bfloat16),
                preferred_element_type=jnp.float32,
            )
            rs_start(pi, 0)

        for t in range(3):
            for pi in (0, 1, 2):
                rs_finish(pi, t)
                if t < 2:
                    rs_start(pi, t + 1)
                else:
                    ag_start(pi, 0)

        for t in range(3):
            for pi in (0, 1, 2):
                inflight.pop(("ag", pi, t)).wait()
                if t < 2:
                    ag_start(pi, t + 1)

    return pl.pallas_call(
        body,
        out_shape=jax.ShapeDtypeStruct((M, N), jnp.bfloat16),
        in_specs=[
            pl.BlockSpec(memory_space=pltpu.VMEM),
            pl.BlockSpec(memory_space=pltpu.VMEM),
        ],
        out_specs=pl.BlockSpec(memory_space=pltpu.VMEM),
        scratch_shapes=[
            pltpu.VMEM((M, N), jnp.float32),
            pltpu.VMEM((896, N), jnp.bfloat16),
            pltpu.VMEM((896, N), jnp.bfloat16),
            pltpu.SemaphoreType.DMA((18,)),
            pltpu.SemaphoreType.DMA((18,)),
        ],
        compiler_params=pltpu.CompilerParams(collective_id=0),
    )(A, B)


# device time: 33547 ns/iter; 1.1838x vs baseline; 1.0314x over previous
import jax
import jax.numpy as jnp
from jax import lax
from jax.experimental import pallas as pl
from jax.experimental.pallas import tpu as pltpu

M = 1024
N = 1024


def kernel(A, B):
    def body(a_ref, b_ref, out_ref, b16_ref, rbuf_ref, send_sems, recv_sems):
        p = lax.axis_index("i")
        bit0 = p % 2
        bit1 = (p // 2) % 2
        bit2 = (p // 4) % 2

        barrier = pltpu.get_barrier_semaphore()
        for mask in (3, 1, 4):
            pl.semaphore_signal(
                barrier,
                inc=1,
                device_id=(lax.bitwise_xor(p, mask),),
                device_id_type=pl.DeviceIdType.MESH,
            )

        parts = []
        for base, rows, masks, bits, soff, sem0 in (
            (0, 384, (3, 1, 4), (bit1, bit0, bit2), (0, 192, 288), 0),
            (384, 384, (1, 4, 3), (bit0 ^ bit1, bit2, bit0), (336, 528, 624), 6),
            (768, 256, (4, 3, 1), (bit2, bit1, bit0), (672, 800, 864), 12),
        ):
            sizes = (rows // 2, rows // 4, rows // 8)
            keep, send = [], []
            cur = base
            for t in range(3):
                sz, b = sizes[t], bits[t]
                keep.append(cur + b * sz)
                send.append(cur + (1 - b) * sz)
                cur = keep[t]
            parts.append(dict(
                sizes=sizes, masks=masks, keep=keep, send=send,
                soff=soff, sem0=sem0,
            ))

        inflight = {}

        def rs_start(pi, t):
            pt = parts[pi]
            sz, off = pt["sizes"][t], pt["soff"][t]
            rdma = pltpu.make_async_remote_copy(
                src_ref=out_ref.at[pl.ds(pt["send"][t], sz), :],
                dst_ref=rbuf_ref.at[pl.ds(off, sz), :],
                send_sem=send_sems.at[pt["sem0"] + t],
                recv_sem=recv_sems.at[pt["sem0"] + t],
                device_id=(lax.bitwise_xor(p, pt["masks"][t]),),
                device_id_type=pl.DeviceIdType.MESH,
            )
            rdma.start()
            inflight[("rs", pi, t)] = rdma

        def rs_finish(pi, t):
            pt = parts[pi]
            sz, off = pt["sizes"][t], pt["soff"][t]
            inflight.pop(("rs", pi, t)).wait()
            summed = (
                out_ref[pl.ds(pt["keep"][t], sz), :]
                + rbuf_ref[pl.ds(off, sz), :]
            )
            if t == 2:
                summed = jnp.maximum(summed, 0.0)
            out_ref[pl.ds(pt["keep"][t], sz), :] = summed

        def ag_start(pi, t):
            pt = parts[pi]
            sz, sbase = pt["sizes"][2 - t], pt["keep"][2 - t]
            rdma = pltpu.make_async_remote_copy(
                src_ref=out_ref.at[pl.ds(sbase, sz), :],
                dst_ref=out_ref.at[pl.ds(sbase, sz), :],
                send_sem=send_sems.at[pt["sem0"] + 3 + t],
                recv_sem=recv_sems.at[pt["sem0"] + 3 + t],
                device_id=(lax.bitwise_xor(p, pt["masks"][2 - t]),),
                device_id_type=pl.DeviceIdType.MESH,
            )
            rdma.start()
            inflight[("ag", pi, t)] = rdma

        b16_ref[:, :] = b_ref[:, :].astype(jnp.bfloat16)

        for pi, (lo, hi) in enumerate(((0, 384), (384, 768), (768, M))):
            out_ref[lo:hi, :] = jnp.dot(
                a_ref[lo:hi, :].astype(jnp.bfloat16),
                b16_ref[:, :],
                preferred_element_type=jnp.float32,
            ).astype(jnp.bfloat16)
            if pi == 0:
                pl.semaphore_wait(barrier, 3)
            rs_start(pi, 0)

        for t in range(3):
            for pi in (0, 1, 2):
                rs_finish(pi, t)
                if t < 2:
                    rs_start(pi, t + 1)
                else:
                    ag_start(pi, 0)

        for t in range(3):
            for pi in (0, 1, 2):
                inflight.pop(("ag", pi, t)).wait()
                if t < 2:
                    ag_start(pi, t + 1)

    return pl.pallas_call(
        body,
        out_shape=jax.ShapeDtypeStruct((M, N), jnp.bfloat16),
        in_specs=[
            pl.BlockSpec(memory_space=pltpu.VMEM),
            pl.BlockSpec(memory_space=pltpu.VMEM),
        ],
        out_specs=pl.BlockSpec(memory_space=pltpu.VMEM),
        scratch_shapes=[
            pltpu.VMEM((512, N), jnp.bfloat16),
            pltpu.VMEM((896, N), jnp.bfloat16),
            pltpu.SemaphoreType.DMA((18,)),
            pltpu.SemaphoreType.DMA((18,)),
        ],
        compiler_params=pltpu.CompilerParams(collective_id=0),
    )(A, B)


# device time: 29510 ns/iter; 1.3457x vs baseline; 1.1368x over previous
import jax
import jax.numpy as jnp
from jax import lax
from jax.experimental import pallas as pl
from jax.experimental.pallas import tpu as pltpu

M = 1024
N = 1024
HALF = N // 2


def kernel(A, B):
    def body(a_ref, b_ref, out_ref, b16_ref, rbuf_ref, send_sems, recv_sems):
        p = lax.axis_index("i")
        bit0 = p % 2
        bit1 = (p // 2) % 2
        bit2 = (p // 4) % 2

        barrier = pltpu.get_barrier_semaphore()
        for mask in (3, 1, 4):
            pl.semaphore_signal(
                barrier,
                inc=1,
                device_id=(lax.bitwise_xor(p, mask),),
                device_id_type=pl.DeviceIdType.MESH,
            )

        row_parts = (
            (0, 384, (3, 1, 4), (bit1, bit0, bit2), (0, 192, 288)),
            (384, 384, (1, 4, 3), (bit0 ^ bit1, bit2, bit0), (336, 528, 624)),
            (768, 256, (4, 3, 1), (bit2, bit1, bit0), (672, 800, 864)),
        )
        parts = []
        for col in (0, HALF):
            for base, rows, masks, bits, soff in row_parts:
                sizes = (rows // 2, rows // 4, rows // 8)
                keep, send = [], []
                cur = base
                for t in range(3):
                    sz, b = sizes[t], bits[t]
                    keep.append(cur + b * sz)
                    send.append(cur + (1 - b) * sz)
                    cur = keep[t]
                parts.append(dict(
                    sizes=sizes, masks=masks, keep=keep, send=send,
                    soff=soff, col=col, sem0=6 * len(parts),
                ))

        inflight = {}

        def rs_start(pi, t):
            pt = parts[pi]
            sz, off, col = pt["sizes"][t], pt["soff"][t], pt["col"]
            rdma = pltpu.make_async_remote_copy(
                src_ref=out_ref.at[pl.ds(pt["send"][t], sz), pl.ds(col, HALF)],
                dst_ref=rbuf_ref.at[pl.ds(off, sz), pl.ds(col, HALF)],
                send_sem=send_sems.at[pt["sem0"] + t],
                recv_sem=recv_sems.at[pt["sem0"] + t],
                device_id=(lax.bitwise_xor(p, pt["masks"][t]),),
                device_id_type=pl.DeviceIdType.MESH,
            )
            rdma.start()
            inflight[("rs", pi, t)] = rdma

        def rs_finish(pi, t):
            pt = parts[pi]
            sz, off, col = pt["sizes"][t], pt["soff"][t], pt["col"]
            inflight.pop(("rs", pi, t)).wait()
            summed = (
                out_ref[pl.ds(pt["keep"][t], sz), pl.ds(col, HALF)]
                + rbuf_ref[pl.ds(off, sz), pl.ds(col, HALF)]
            )
            if t == 2:
                summed = jnp.maximum(summed, 0.0)
            out_ref[pl.ds(pt["keep"][t], sz), pl.ds(col, HALF)] = summed

        def ag_start(pi, t):
            pt = parts[pi]
            sz, sbase, col = pt["sizes"][2 - t], pt["keep"][2 - t], pt["col"]
            rdma = pltpu.make_async_remote_copy(
                src_ref=out_ref.at[pl.ds(sbase, sz), pl.ds(col, HALF)],
                dst_ref=out_ref.at[pl.ds(sbase, sz), pl.ds(col, HALF)],
                send_sem=send_sems.at[pt["sem0"] + 3 + t],
                recv_sem=recv_sems.at[pt["sem0"] + 3 + t],
                device_id=(lax.bitwise_xor(p, pt["masks"][2 - t]),),
                device_id_type=pl.DeviceIdType.MESH,
            )
            rdma.start()
            inflight[("ag", pi, t)] = rdma

        b16_ref[:, :] = b_ref[:, :].astype(jnp.bfloat16)

        for pi, (lo, hi) in enumerate(((0, 384), (384, 768), (768, M))):
            out_ref[lo:hi, :] = jnp.dot(
                a_ref[lo:hi, :].astype(jnp.bfloat16),
                b16_ref[:, :],
                preferred_element_type=jnp.float32,
            ).astype(jnp.bfloat16)
            if pi == 0:
                pl.semaphore_wait(barrier, 3)
            rs_start(pi, 0)
            rs_start(pi + 3, 0)

        for t in range(3):
            for pi in range(6):
                rs_finish(pi, t)
                if t < 2:
                    rs_start(pi, t + 1)
                else:
                    ag_start(pi, 0)

        for t in range(3):
            for pi in range(6):
                inflight.pop(("ag", pi, t)).wait()
                if t < 2:
                    ag_start(pi, t + 1)

    return pl.pallas_call(
        body,
        out_shape=jax.ShapeDtypeStruct((M, N), jnp.bfloat16),
        in_specs=[
            pl.BlockSpec(memory_space=pltpu.VMEM),
            pl.BlockSpec(memory_space=pltpu.VMEM),
        ],
        out_specs=pl.BlockSpec(memory_space=pltpu.VMEM),
        scratch_shapes=[
            pltpu.VMEM((512, N), jnp.bfloat16),
            pltpu.VMEM((896, N), jnp.bfloat16),
            pltpu.SemaphoreType.DMA((36,)),
            pltpu.SemaphoreType.DMA((36,)),
        ],
        compiler_params=pltpu.CompilerParams(collective_id=0),
    )(A, B)


# device time: 27589 ns/iter; 1.4395x vs baseline; 1.0696x over previous
import jax
import jax.numpy as jnp
from jax import lax
from jax.experimental import pallas as pl
from jax.experimental.pallas import tpu as pltpu

M = 1024
N = 1024
HALF = N // 2


def _parity(a, j):
    return bin(a & j).count("1") % 2


def kernel(A, B):
    def body(a_ref, b_ref, out_ref, b16_ref, rbuf_ref, send_sems, recv_sems):
        p = lax.axis_index("i")
        bit0 = p % 2
        bit1 = (p // 2) % 2
        bit2 = (p // 4) % 2

        barrier = pltpu.get_barrier_semaphore()
        for mask in (3, 1, 4):
            pl.semaphore_signal(
                barrier,
                inc=1,
                device_id=(lax.bitwise_xor(p, mask),),
                device_id_type=pl.DeviceIdType.MESH,
            )

        row_parts = (
            (0, 384, 3, bit1, (1, 4, 5), (1, 4), (0, 48, 96), 384),
            (384, 384, 1, bit0 ^ bit1, (4, 3, 7), (4, 1), (144, 192, 240), 576),
            (768, 256, 4, bit2, (3, 1, 2), (2, 3), (288, 320, 352), 768),
        )
        bits_of = {1: bit0, 2: bit1, 3: bit0 ^ bit1, 4: bit2}

        parts = []
        for col in (0, HALF):
            for base, rows, m0, chi0, js, (a1, a2), soff, r1off in row_parts:
                half, quart, eighth = rows // 2, rows // 4, rows // 8
                keep0 = base + chi0 * half
                send0 = base + (1 - chi0) * half
                chi1 = bits_of[a1]
                chi2 = bits_of[a2]
                my8 = keep0 + chi1 * quart + chi2 * eighth
                peer8 = [
                    keep0
                    + (chi1 ^ _parity(a1, j)) * quart
                    + (chi2 ^ _parity(a2, j)) * eighth
                    for j in js
                ]
                parts.append(dict(
                    rows=rows, m0=m0, js=js, col=col,
                    keep0=keep0, send0=send0, half=half, eighth=eighth,
                    my8=my8, peer8=peer8, soff=soff, r1off=r1off,
                    sem0=8 * len(parts),
                ))

        inflight = {}

        def copy(pi, key, sem, src, dst, mask):
            pt = parts[pi]
            rdma = pltpu.make_async_remote_copy(
                src_ref=src,
                dst_ref=dst,
                send_sem=send_sems.at[pt["sem0"] + sem],
                recv_sem=recv_sems.at[pt["sem0"] + sem],
                device_id=(lax.bitwise_xor(p, mask),),
                device_id_type=pl.DeviceIdType.MESH,
            )
            rdma.start()
            inflight[(key, pi)] = rdma

        def rs1_start(pi):
            pt = parts[pi]
            sz, col = pt["half"], pt["col"]
            copy(
                pi, "rs1", 0,
                out_ref.at[pl.ds(pt["send0"], sz), pl.ds(col, HALF)],
                rbuf_ref.at[pl.ds(pt["r1off"], sz), pl.ds(col, HALF)],
                pt["m0"],
            )

        def rs1_finish_and_direct(pi):
            pt = parts[pi]
            sz, col = pt["half"], pt["col"]
            inflight.pop(("rs1", pi)).wait()
            out_ref[pl.ds(pt["keep0"], sz), pl.ds(col, HALF)] = (
                out_ref[pl.ds(pt["keep0"], sz), pl.ds(col, HALF)]
                + rbuf_ref[pl.ds(pt["r1off"], sz), pl.ds(col, HALF)]
            )
            e = pt["eighth"]
            for gi, j in enumerate(pt["js"]):
                copy(
                    pi, f"drs{gi}", 1 + gi,
                    out_ref.at[pl.ds(pt["peer8"][gi], e), pl.ds(col, HALF)],
                    rbuf_ref.at[pl.ds(pt["soff"][gi], e), pl.ds(col, HALF)],
                    j,
                )

        def direct_finish_and_ag(pi):
            pt = parts[pi]
            e, col = pt["eighth"], pt["col"]
            for gi in range(3):
                inflight.pop((f"drs{gi}", pi)).wait()
            out_ref[pl.ds(pt["my8"], e), pl.ds(col, HALF)] = jnp.maximum(
                out_ref[pl.ds(pt["my8"], e), pl.ds(col, HALF)]
                + rbuf_ref[pl.ds(pt["soff"][0], e), pl.ds(col, HALF)]
                + rbuf_ref[pl.ds(pt["soff"][1], e), pl.ds(col, HALF)]
                + rbuf_ref[pl.ds(pt["soff"][2], e), pl.ds(col, HALF)],
                0.0,
            )
            for gi, j in enumerate(pt["js"]):
                copy(
                    pi, f"dag{gi}", 4 + gi,
                    out_ref.at[pl.ds(pt["my8"], e), pl.ds(col, HALF)],
                    out_ref.at[pl.ds(pt["my8"], e), pl.ds(col, HALF)],
                    j,
                )

        def dag_finish_and_ag2(pi):
            pt = parts[pi]
            sz, col = pt["half"], pt["col"]
            for gi in range(3):
                inflight.pop((f"dag{gi}", pi)).wait()
            copy(
                pi, "ag2", 7,
                out_ref.at[pl.ds(pt["keep0"], sz), pl.ds(col, HALF)],
                out_ref.at[pl.ds(pt["keep0"], sz), pl.ds(col, HALF)],
                pt["m0"],
            )

        b16_ref[:, :] = b_ref[:, :].astype(jnp.bfloat16)

        for pi, (lo, hi) in enumerate(((0, 384), (384, 768), (768, M))):
            out_ref[lo:hi, :] = jnp.dot(
                a_ref[lo:hi, :].astype(jnp.bfloat16),
                b16_ref[:, :],
                preferred_element_type=jnp.float32,
            ).astype(jnp.bfloat16)
            if pi == 0:
                pl.semaphore_wait(barrier, 3)
            rs1_start(pi)
            rs1_start(pi + 3)

        for pi in range(6):
            rs1_finish_and_direct(pi)
        for pi in range(6):
            direct_finish_and_ag(pi)
        for pi in range(6):
            dag_finish_and_ag2(pi)
        for pi in range(6):
            inflight.pop(("ag2", pi)).wait()

    return pl.pallas_call(
        body,
        out_shape=jax.ShapeDtypeStruct((M, N), jnp.bfloat16),
        in_specs=[
            pl.BlockSpec(memory_space=pltpu.VMEM),
            pl.BlockSpec(memory_space=pltpu.VMEM),
        ],
        out_specs=pl.BlockSpec(memory_space=pltpu.VMEM),
        scratch_shapes=[
            pltpu.VMEM((512, N), jnp.bfloat16),
            pltpu.VMEM((896, N), jnp.bfloat16),
            pltpu.SemaphoreType.DMA((48,)),
            pltpu.SemaphoreType.DMA((48,)),
        ],
        compiler_params=pltpu.CompilerParams(collective_id=0),
    )(A, B)
